# baseline (speedup 1.0000x reference)
import functools

import jax
import jax.numpy as jnp
from jax import lax
from jax.experimental import pallas as pl
from jax.experimental.pallas import tpu as pltpu
from jax.experimental.pallas import tpu_sc as plsc

EMBED_DIM = 32


@functools.cache
def _make_gather(n_b: int, n_s: int, vocab: int):
    info = plsc.get_sparse_core_info()
    nc, ns = info.num_cores, info.num_subcores
    nw = nc * ns
    bpw = n_b // nw
    n_tb = bpw // 128
    n_te = EMBED_DIM // 8
    assert n_b % (nw * 128) == 0

    mesh = plsc.VectorSubcoreMesh(core_axis_name="c", subcore_axis_name="s")

    @functools.partial(
        pl.kernel,
        mesh=mesh,
        out_type=jax.ShapeDtypeStruct(
            (n_s, n_te, n_b // 128, 8, 128), jnp.float32
        ),
        scratch_types=[
            pltpu.VMEM((n_s, bpw), jnp.int32),
            pltpu.VMEM((2, bpw, EMBED_DIM), jnp.float32),
            pltpu.VMEM((2, n_te, n_tb, 8, 128), jnp.float32),
            [pltpu.SemaphoreType.DMA] * 2,
            [pltpu.SemaphoreType.DMA] * 2,
        ],
        compiler_params=pltpu.CompilerParams(
            use_tc_tiling_on_sc=False, needs_layout_passes=False
        ),
    )
    def gather_kernel(idx_hbm, table_hbm, out_hbm, idx_v, rows_v, tbuf_v, gsems, ssems):
        wid = lax.axis_index("s") * nc + lax.axis_index("c")
        b0 = wid * bpw
        tb0 = wid * n_tb
        pltpu.sync_copy(idx_hbm.at[:, pl.ds(b0, bpw)], idx_v)
        iota = lax.iota(jnp.int32, 16)

        def start_gather(s):
            return pltpu.async_copy(
                table_hbm.at[idx_v.at[s]], rows_v.at[s % 2], gsems[s % 2]
            )

        def store_copies(s):
            bb = s % 2
            return [
                (tbuf_v.at[bb, te], out_hbm.at[s, te, pl.ds(tb0, n_tb)], ssems[bb])
                for te in range(n_te)
            ]

        def transpose(s):
            bb = s % 2

            def body(g, carry):
                te = g >> 5
                tbl = (g >> 3) & 3
                e8 = g & 7
                colv = jnp.full((16,), te * 8 + e8, jnp.int32)
                for j in range(8):
                    row_ids = tbl * 128 + j * 16 + iota
                    v = plsc.load_gather(rows_v.at[bb], [row_ids, colv])
                    tbuf_v[bb, te, tbl, e8, pl.ds(j * 16, 16)] = v
                return carry

            lax.fori_loop(0, n_te * n_tb * 8, body, 0)

        gathers = [None] * n_s
        gathers[0] = start_gather(0)
        for s in range(n_s):
            if s + 1 < n_s:
                gathers[s + 1] = start_gather(s + 1)
            gathers[s].wait()
            if s >= 2:
                for args in store_copies(s - 2):
                    pltpu.make_async_copy(*args).wait()
            transpose(s)
            for args in store_copies(s):
                pltpu.async_copy(*args)
        for s in (n_s - 2, n_s - 1):
            for args in store_copies(s):
                pltpu.make_async_copy(*args).wait()

    return gather_kernel


def kernel(inputs, embed_table):
    b, s = inputs.shape
    idx_t = inputs.T.astype(jnp.int32)
    out5 = _make_gather(b, s, embed_table.shape[0])(idx_t, embed_table)
    return out5.transpose(2, 4, 0, 1, 3).reshape(b, s, EMBED_DIM)
